# baseline (speedup 1.0000x reference)
import functools

import jax
import jax.numpy as jnp
from jax import lax
from jax.experimental import pallas as pl
from jax.experimental.pallas import tpu as pltpu
from jax.experimental.pallas import tpu_sc as plsc

L = 5
G = 64
N = 10000
E = 320000
D = 128
NW = 32
NPW = 320
NPAD = NW * NPW
CHUNK = 80
NLANE = 16


def _bn_n(y, g, b):
    n = y.shape[0]
    mu = jnp.mean(y, axis=0, keepdims=True)
    dd = y - mu
    dd2 = dd * dd
    var = ((jnp.sum(dd2[:n // 2], axis=0, keepdims=True)
            + jnp.sum(dd2[n // 2:], axis=0, keepdims=True)) * (1.0 / n))
    return dd * lax.rsqrt(var + 1e-5) * g + b


def _bn_g(y, g, b):
    mu = jnp.mean(y, axis=0, keepdims=True)
    dd = y - mu
    var = jnp.mean(dd * dd, axis=0, keepdims=True)
    return dd * lax.rsqrt(var + 1e-5) * g + b


def _dot(a, b):
    return jnp.dot(a, b, preferred_element_type=jnp.float32,
                   precision=lax.Precision.DEFAULT)


def _dot_hi(a, b):
    return jnp.dot(a, b, preferred_element_type=jnp.float32,
                   precision=lax.Precision.HIGHEST)


def _init_body(x_ref, v_ref, o_ref):
    o_ref[...] = x_ref[...] + v_ref[...]


def _mid_body(h_in_ref, agg_ref, segp_ref, vn_ref, bn1_ref,
              w1_ref, b1_ref, g1_ref, be1_ref, w2_ref, b2_ref, eps_ref,
              bg_ref, bb_ref,
              vw1_ref, vb1_ref, vg1_ref, vbe1_ref,
              vw2_ref, vb2_ref, vg2_ref, vbe2_ref,
              hout_ref, vnout_ref):
    h_in = h_in_ref[...]
    h = (1.0 + eps_ref[0, 0]) * h_in + agg_ref[...]
    y = _dot(h, w1_ref[...]) + b1_ref[...]
    y = jnp.maximum(_bn_n(y, g1_ref[...], be1_ref[...]), 0.0)
    z = _dot(y, w2_ref[...]) + b2_ref[...]
    hc = jnp.maximum(_bn_n(z, bg_ref[...], bb_ref[...]), 0.0)

    seg = segp_ref[0]
    for w in range(1, NW):
        seg = seg + segp_ref[w]
    vtmp = seg + vn_ref[...]
    t = _dot(vtmp, vw1_ref[...]) + vb1_ref[...]
    t = jnp.maximum(_bn_g(t, vg1_ref[...], vbe1_ref[...]), 0.0)
    t = _dot(t, vw2_ref[...]) + vb2_ref[...]
    vn_next = jnp.maximum(_bn_g(t, vg2_ref[...], vbe2_ref[...]), 0.0)
    vnout_ref[...] = vn_next

    oh_ng = (bn1_ref[...]
             == lax.broadcasted_iota(jnp.int32, (1, G), 1)).astype(jnp.float32)
    hout_ref[...] = hc + _dot_hi(oh_ng, vn_next)


def _last_body(h_in_ref, agg_ref, b1n_ref,
               w1_ref, b1_ref, g1_ref, be1_ref, w2_ref, b2_ref, eps_ref,
               bg_ref, bb_ref, pw_ref, pb_ref, out_ref):
    h_in = h_in_ref[...]
    h = (1.0 + eps_ref[0, 0]) * h_in + agg_ref[...]
    y = _dot(h, w1_ref[...]) + b1_ref[...]
    y = jnp.maximum(_bn_n(y, g1_ref[...], be1_ref[...]), 0.0)
    z = _dot(y, w2_ref[...]) + b2_ref[...]
    hc = _bn_n(z, bg_ref[...], bb_ref[...])

    oh_gn = (lax.broadcasted_iota(jnp.int32, (G, 1), 0)
             == b1n_ref[...]).astype(jnp.float32)
    counts = jnp.sum(oh_gn, axis=1, keepdims=True)
    hg = _dot_hi(oh_gn, hc) / jnp.maximum(counts, 1.0)
    out_ref[...] = _dot(hg, pw_ref[...]) + pb_ref[...]


def _make_edge_agg():
    mesh = plsc.VectorSubcoreMesh(core_axis_name="c", subcore_axis_name="s")
    f32, i32 = jnp.float32, jnp.int32

    @functools.partial(
        pl.kernel, mesh=mesh,
        compiler_params=pltpu.CompilerParams(needs_layout_passes=False),
        out_type=[jax.ShapeDtypeStruct((NPAD, D), f32),
                  jax.ShapeDtypeStruct((NW, G, D), f32)],
        scratch_types=[
            pltpu.VMEM((NPW, D), f32),
            pltpu.VMEM((G, D), f32),
            pltpu.VMEM((CHUNK, D), f32),
            pltpu.VMEM((CHUNK, D), f32),
            pltpu.VMEM((CHUNK,), i32),
            pltpu.VMEM((CHUNK,), i32),
            pltpu.VMEM((CHUNK,), i32),
            pltpu.VMEM((CHUNK,), i32),
            pltpu.VMEM((NLANE,), i32),
            pltpu.SemaphoreType.DMA,
            pltpu.SemaphoreType.DMA,
        ],
    )
    def edge_agg(h_hbm, ea_hbm, srcs_hbm, perm_hbm, dsts_hbm, rp_hbm,
                 bat_hbm, agg_out, segp_out,
                 aggb, segb, hbuf, eabuf, sbuf, pbuf, dbuf, bbuf, rpv,
                 sem0, sem1):
        cid = lax.axis_index("c")
        sid = lax.axis_index("s")
        w = sid * 2 + cid
        ns = w * NPW
        ne = ns + NPW
        zero = jnp.zeros((NLANE,), f32)
        lanes = lax.iota(i32, NLANE)

        def zrow(r, c):
            for j in range(D // NLANE):
                aggb[r, pl.ds(j * NLANE, NLANE)] = zero
            return c
        lax.fori_loop(0, NPW, zrow, 0)

        def zrow2(r, c):
            for j in range(D // NLANE):
                segb[r, pl.ds(j * NLANE, NLANE)] = zero
            return c
        lax.fori_loop(0, G, zrow2, 0)

        def rp_at(idx):
            pltpu.sync_copy(rp_hbm.at[pl.ds(idx, NLANE)], rpv)
            return rpv[...][0]

        estart = rp_at(ns)
        eend = rp_at(ne)
        base = (estart // 8) * 8
        nchunks = lax.max((eend - base + (CHUNK - 1)) // CHUNK, 0)

        def echunk(k, c):
            eb = base + k * CHUNK
            pltpu.sync_copy(srcs_hbm.at[pl.ds(eb, CHUNK)], sbuf)
            pltpu.sync_copy(perm_hbm.at[pl.ds(eb, CHUNK)], pbuf)
            pltpu.sync_copy(dsts_hbm.at[pl.ds(eb, CHUNK)], dbuf)
            cp0 = pltpu.async_copy(h_hbm.at[sbuf], hbuf, sem0)
            cp1 = pltpu.async_copy(ea_hbm.at[pbuf], eabuf, sem1)
            cp0.wait()
            cp1.wait()
            dvs, msks = [], []
            for g in range(CHUNK // NLANE):
                dv = dbuf[pl.ds(g * NLANE, NLANE)] - ns
                ea_ids = eb + g * NLANE + lanes
                msk = (ea_ids >= estart) & (ea_ids < eend)
                dvs.append(dv)
                msks.append(msk)

            def fcol(f, c2):
                fs = jnp.full((NLANE,), f, i32)
                for g in range(CHUNK // NLANE):
                    el = lanes + g * NLANE
                    hv = plsc.load_gather(hbuf, [el, fs])
                    ev = plsc.load_gather(eabuf, [el, fs])
                    v = jnp.maximum(hv + ev, 0.0)
                    plsc.addupdate_scatter(aggb, [dvs[g], fs], v,
                                           mask=msks[g])
                return c2
            lax.fori_loop(0, D, fcol, 0)
            return c
        lax.fori_loop(0, nchunks, echunk, 0)
        pltpu.sync_copy(aggb.at[pl.ds(0, NPW)], agg_out.at[pl.ds(ns, NPW)])

        sbase = (ns // 8) * 8
        for k in range(NPW // CHUNK):
            nb = sbase + k * CHUNK
            pltpu.sync_copy(bat_hbm.at[pl.ds(nb, CHUNK)], bbuf)
            pltpu.sync_copy(h_hbm.at[pl.ds(nb, CHUNK)], hbuf)
            gvs, msks = [], []
            for g in range(CHUNK // NLANE):
                gv = bbuf[pl.ds(g * NLANE, NLANE)]
                n_ids = nb + g * NLANE + lanes
                msk = (n_ids >= ns) & (n_ids < ne) & (n_ids < N)
                gvs.append(gv)
                msks.append(msk)

            def scol(f, c2):
                fs = jnp.full((NLANE,), f, i32)
                for g in range(CHUNK // NLANE):
                    el = lanes + g * NLANE
                    hv = plsc.load_gather(hbuf, [el, fs])
                    plsc.addupdate_scatter(segb, [gvs[g], fs], hv,
                                           mask=msks[g])
                return c2
            lax.fori_loop(0, D, scol, 0)
        pltpu.sync_copy(segb, segp_out.at[w])

    return edge_agg


def kernel(x, edge_attr, params, edge_index, batch):
    f32, i32 = jnp.float32, jnp.int32
    T = params['pred_b'].shape[0]
    src = edge_index[0]
    dst = edge_index[1]

    perm = jnp.argsort(dst, stable=True).astype(i32)
    dst_s = dst[perm]
    src_s = src[perm]
    rowptr = jnp.searchsorted(dst_s, jnp.arange(NPAD + NLANE, dtype=i32),
                              side='left').astype(i32)
    pad_e = jnp.zeros((CHUNK,), i32)
    src_p = jnp.concatenate([src_s, pad_e])
    perm_p = jnp.concatenate([perm, pad_e])
    dst_p = jnp.concatenate([dst_s, pad_e])
    bat_p = jnp.concatenate([batch.astype(i32),
                             jnp.zeros((NPAD + CHUNK - N,), i32)])
    bn1 = batch.reshape(N, 1)
    b1n = batch.reshape(1, N)

    init_call = pl.pallas_call(
        _init_body, out_shape=jax.ShapeDtypeStruct((N, D), f32))
    h_in = init_call(x, params['vn_emb'].reshape(1, D))
    vn = jnp.broadcast_to(params['vn_emb'], (G, D))

    edge_agg = _make_edge_agg()
    mid_call = pl.pallas_call(
        _mid_body,
        out_shape=[jax.ShapeDtypeStruct((N, D), f32),
                   jax.ShapeDtypeStruct((G, D), f32)])
    last_call = pl.pallas_call(
        _last_body, out_shape=jax.ShapeDtypeStruct((G, T), f32))

    for l in range(L):
        p = params['convs'][l]
        bn = params['bns'][l]
        h_pad = jnp.concatenate([h_in, jnp.zeros((NPAD - N, D), f32)])
        agg_pad, segp = edge_agg(h_pad, edge_attr, src_p, perm_p, dst_p,
                                 rowptr, bat_p)
        agg = agg_pad[:N]
        common = (p['w1'], p['b1'].reshape(1, -1), p['g1'].reshape(1, -1),
                  p['be1'].reshape(1, -1), p['w2'], p['b2'].reshape(1, -1),
                  p['eps'].reshape(1, 1),
                  bn['g'].reshape(1, -1), bn['b'].reshape(1, -1))
        if l < L - 1:
            q = params['vn_mlps'][l]
            h_in, vn = mid_call(
                h_in, agg, segp, vn, bn1, *common,
                q['w1'], q['b1'].reshape(1, -1), q['g1'].reshape(1, -1),
                q['be1'].reshape(1, -1),
                q['w2'], q['b2'].reshape(1, -1), q['g2'].reshape(1, -1),
                q['be2'].reshape(1, -1))
        else:
            out = last_call(h_in, agg, b1n, *common,
                            params['pred_w'], params['pred_b'].reshape(1, -1))
    return out


# speedup vs baseline: 1.0942x; 1.0942x over previous
import functools

import jax
import jax.numpy as jnp
from jax import lax
from jax.experimental import pallas as pl
from jax.experimental.pallas import tpu as pltpu
from jax.experimental.pallas import tpu_sc as plsc

L = 5
G = 64
N = 10000
E = 320000
D = 128
NW = 32
NPW = 320
NPAD = NW * NPW
CHUNK = 80
IDXCAP = 11280
NLANE = 16


def _bn_n(y, g, b):
    n = y.shape[0]
    mu = jnp.mean(y, axis=0, keepdims=True)
    dd = y - mu
    dd2 = dd * dd
    var = ((jnp.sum(dd2[:n // 2], axis=0, keepdims=True)
            + jnp.sum(dd2[n // 2:], axis=0, keepdims=True)) * (1.0 / n))
    return dd * lax.rsqrt(var + 1e-5) * g + b


def _bn_g(y, g, b):
    mu = jnp.mean(y, axis=0, keepdims=True)
    dd = y - mu
    var = jnp.mean(dd * dd, axis=0, keepdims=True)
    return dd * lax.rsqrt(var + 1e-5) * g + b


def _dot(a, b):
    return jnp.dot(a, b, preferred_element_type=jnp.float32,
                   precision=lax.Precision.DEFAULT)


def _dot_hi(a, b):
    return jnp.dot(a, b, preferred_element_type=jnp.float32,
                   precision=lax.Precision.HIGHEST)


def _init_body(x_ref, v_ref, o_ref):
    o_ref[...] = x_ref[...] + v_ref[...]


def _mid_body(h_in_ref, agg_ref, segp_ref, vn_ref, bn1_ref,
              w1_ref, b1_ref, g1_ref, be1_ref, w2_ref, b2_ref, eps_ref,
              bg_ref, bb_ref,
              vw1_ref, vb1_ref, vg1_ref, vbe1_ref,
              vw2_ref, vb2_ref, vg2_ref, vbe2_ref,
              hout_ref, vnout_ref):
    h_in = h_in_ref[...]
    h = (1.0 + eps_ref[0, 0]) * h_in + agg_ref[...]
    y = _dot(h, w1_ref[...]) + b1_ref[...]
    y = jnp.maximum(_bn_n(y, g1_ref[...], be1_ref[...]), 0.0)
    z = _dot(y, w2_ref[...]) + b2_ref[...]
    hc = jnp.maximum(_bn_n(z, bg_ref[...], bb_ref[...]), 0.0)

    seg = segp_ref[0]
    for w in range(1, NW):
        seg = seg + segp_ref[w]
    vtmp = seg + vn_ref[...]
    t = _dot(vtmp, vw1_ref[...]) + vb1_ref[...]
    t = jnp.maximum(_bn_g(t, vg1_ref[...], vbe1_ref[...]), 0.0)
    t = _dot(t, vw2_ref[...]) + vb2_ref[...]
    vn_next = jnp.maximum(_bn_g(t, vg2_ref[...], vbe2_ref[...]), 0.0)
    vnout_ref[...] = vn_next

    oh_ng = (bn1_ref[...]
             == lax.broadcasted_iota(jnp.int32, (1, G), 1)).astype(jnp.float32)
    hout_ref[...] = hc + _dot_hi(oh_ng, vn_next)


def _last_body(h_in_ref, agg_ref, b1n_ref,
               w1_ref, b1_ref, g1_ref, be1_ref, w2_ref, b2_ref, eps_ref,
               bg_ref, bb_ref, pw_ref, pb_ref, out_ref):
    h_in = h_in_ref[...]
    h = (1.0 + eps_ref[0, 0]) * h_in + agg_ref[...]
    y = _dot(h, w1_ref[...]) + b1_ref[...]
    y = jnp.maximum(_bn_n(y, g1_ref[...], be1_ref[...]), 0.0)
    z = _dot(y, w2_ref[...]) + b2_ref[...]
    hc = _bn_n(z, bg_ref[...], bb_ref[...])

    oh_gn = (lax.broadcasted_iota(jnp.int32, (G, 1), 0)
             == b1n_ref[...]).astype(jnp.float32)
    counts = jnp.sum(oh_gn, axis=1, keepdims=True)
    hg = _dot_hi(oh_gn, hc) / jnp.maximum(counts, 1.0)
    out_ref[...] = _dot(hg, pw_ref[...]) + pb_ref[...]


def _make_edge_agg():
    mesh = plsc.VectorSubcoreMesh(core_axis_name="c", subcore_axis_name="s")
    f32, i32 = jnp.float32, jnp.int32

    @functools.partial(
        pl.kernel, mesh=mesh,
        compiler_params=pltpu.CompilerParams(needs_layout_passes=False),
        out_type=[jax.ShapeDtypeStruct((NPAD, D), f32),
                  jax.ShapeDtypeStruct((NW, G, D), f32)],
        scratch_types=[
            pltpu.VMEM((NPW, D), f32),
            pltpu.VMEM((G, D), f32),
            pltpu.VMEM((CHUNK, D), f32),
            pltpu.VMEM((CHUNK, D), f32),
            pltpu.VMEM((CHUNK, D), f32),
            pltpu.VMEM((CHUNK, D), f32),
            pltpu.VMEM((IDXCAP,), i32),
            pltpu.VMEM((IDXCAP,), i32),
            pltpu.VMEM((IDXCAP,), i32),
            pltpu.VMEM((CHUNK,), i32),
            pltpu.VMEM((NLANE,), i32),
            pltpu.SemaphoreType.DMA,
            pltpu.SemaphoreType.DMA,
            pltpu.SemaphoreType.DMA,
            pltpu.SemaphoreType.DMA,
        ],
    )
    def edge_agg(h_hbm, ea_hbm, srcs_hbm, perm_hbm, dsts_hbm, rp_hbm,
                 bat_hbm, agg_out, segp_out,
                 aggb, segb, hb0, eb0, hb1, eb1, sbufL, pbufL, dbufL,
                 bbuf, rpv, semh0, seme0, semh1, seme1):
        cid = lax.axis_index("c")
        sid = lax.axis_index("s")
        w = sid * 2 + cid
        ns = w * NPW
        ne = ns + NPW
        zero = jnp.zeros((NLANE,), f32)
        lanes = lax.iota(i32, NLANE)

        def zrow(r, c):
            for j in range(D // NLANE):
                aggb[r, pl.ds(j * NLANE, NLANE)] = zero
            return c
        lax.fori_loop(0, NPW, zrow, 0)

        def zrow2(r, c):
            for j in range(D // NLANE):
                segb[r, pl.ds(j * NLANE, NLANE)] = zero
            return c
        lax.fori_loop(0, G, zrow2, 0)

        def rp_at(idx):
            pltpu.sync_copy(rp_hbm.at[pl.ds(idx, NLANE)], rpv)
            return rpv[...][0]

        estart = rp_at(ns)
        eend = rp_at(ne)
        base = (estart // 8) * 8
        nchunks = lax.max((eend - base + (CHUNK - 1)) // CHUNK, 0)
        nchunks = lax.min(nchunks, IDXCAP // CHUNK)

        pltpu.sync_copy(srcs_hbm.at[pl.ds(base, IDXCAP)], sbufL)
        pltpu.sync_copy(perm_hbm.at[pl.ds(base, IDXCAP)], pbufL)
        pltpu.sync_copy(dsts_hbm.at[pl.ds(base, IDXCAP)], dbufL)

        bufs = ((hb0, eb0, semh0, seme0), (hb1, eb1, semh1, seme1))

        def start(k, b):
            hb, ebuf, semh, seme = bufs[b]
            koff = k * CHUNK
            pltpu.async_copy(h_hbm.at[sbufL.at[pl.ds(koff, CHUNK)]],
                             hb, semh)
            pltpu.async_copy(ea_hbm.at[pbufL.at[pl.ds(koff, CHUNK)]],
                             ebuf, seme)

        def wait(b):
            hb, ebuf, semh, seme = bufs[b]
            pltpu.make_async_copy(h_hbm.at[sbufL.at[pl.ds(0, CHUNK)]],
                                  hb, semh).wait()
            pltpu.make_async_copy(ea_hbm.at[pbufL.at[pl.ds(0, CHUNK)]],
                                  ebuf, seme).wait()

        def compute(k, b):
            hb, ebuf, _, _ = bufs[b]
            eb = base + k * CHUNK
            koff = k * CHUNK
            dvs, msks = [], []
            for g in range(CHUNK // NLANE):
                dv = dbufL[pl.ds(koff + g * NLANE, NLANE)] - ns
                ea_ids = eb + g * NLANE + lanes
                msk = (ea_ids >= estart) & (ea_ids < eend)
                dvs.append(dv)
                msks.append(msk)

            def fcol(f, c2):
                fs = jnp.full((NLANE,), f, i32)
                for g in range(CHUNK // NLANE):
                    el = lanes + g * NLANE
                    hv = plsc.load_gather(hb, [el, fs])
                    ev = plsc.load_gather(ebuf, [el, fs])
                    v = jnp.maximum(hv + ev, 0.0)
                    plsc.addupdate_scatter(aggb, [dvs[g], fs], v,
                                           mask=msks[g])
                return c2
            lax.fori_loop(0, D, fcol, 0)

        @pl.when(nchunks > 0)
        def _():
            start(0, 0)

        def epair(p, c):
            k0 = 2 * p
            k1 = k0 + 1

            @pl.when(k1 < nchunks)
            def _():
                start(k1, 1)
            wait(0)
            compute(k0, 0)

            @pl.when(k1 + 1 < nchunks)
            def _():
                start(k1 + 1, 0)

            @pl.when(k1 < nchunks)
            def _():
                wait(1)
                compute(k1, 1)
            return c
        lax.fori_loop(0, (nchunks + 1) // 2, epair, 0)
        pltpu.sync_copy(aggb.at[pl.ds(0, NPW)], agg_out.at[pl.ds(ns, NPW)])

        sbase = (ns // 8) * 8
        for k in range(NPW // CHUNK):
            nb = sbase + k * CHUNK
            pltpu.sync_copy(bat_hbm.at[pl.ds(nb, CHUNK)], bbuf)
            pltpu.sync_copy(h_hbm.at[pl.ds(nb, CHUNK)], hb0)
            gvs, msks = [], []
            for g in range(CHUNK // NLANE):
                gv = bbuf[pl.ds(g * NLANE, NLANE)]
                n_ids = nb + g * NLANE + lanes
                msk = (n_ids >= ns) & (n_ids < ne) & (n_ids < N)
                gvs.append(gv)
                msks.append(msk)

            def scol(f, c2):
                fs = jnp.full((NLANE,), f, i32)
                for g in range(CHUNK // NLANE):
                    el = lanes + g * NLANE
                    hv = plsc.load_gather(hb0, [el, fs])
                    plsc.addupdate_scatter(segb, [gvs[g], fs], hv,
                                           mask=msks[g])
                return c2
            lax.fori_loop(0, D, scol, 0)
        pltpu.sync_copy(segb, segp_out.at[w])

    return edge_agg


def kernel(x, edge_attr, params, edge_index, batch):
    f32, i32 = jnp.float32, jnp.int32
    T = params['pred_b'].shape[0]
    src = edge_index[0]
    dst = edge_index[1]

    perm = jnp.argsort(dst, stable=True).astype(i32)
    dst_s = dst[perm]
    src_s = src[perm]
    rowptr = jnp.searchsorted(dst_s, jnp.arange(NPAD + NLANE, dtype=i32),
                              side='left').astype(i32)
    pad_e = jnp.zeros((IDXCAP,), i32)
    src_p = jnp.concatenate([src_s, pad_e])
    perm_p = jnp.concatenate([perm, pad_e])
    dst_p = jnp.concatenate([dst_s, pad_e])
    bat_p = jnp.concatenate([batch.astype(i32),
                             jnp.zeros((NPAD + CHUNK - N,), i32)])
    bn1 = batch.reshape(N, 1)
    b1n = batch.reshape(1, N)

    init_call = pl.pallas_call(
        _init_body, out_shape=jax.ShapeDtypeStruct((N, D), f32))
    h_in = init_call(x, params['vn_emb'].reshape(1, D))
    vn = jnp.broadcast_to(params['vn_emb'], (G, D))

    edge_agg = _make_edge_agg()
    mid_call = pl.pallas_call(
        _mid_body,
        out_shape=[jax.ShapeDtypeStruct((N, D), f32),
                   jax.ShapeDtypeStruct((G, D), f32)])
    last_call = pl.pallas_call(
        _last_body, out_shape=jax.ShapeDtypeStruct((G, T), f32))

    for l in range(L):
        p = params['convs'][l]
        bn = params['bns'][l]
        h_pad = jnp.concatenate([h_in, jnp.zeros((NPAD - N, D), f32)])
        agg_pad, segp = edge_agg(h_pad, edge_attr, src_p, perm_p, dst_p,
                                 rowptr, bat_p)
        agg = agg_pad[:N]
        common = (p['w1'], p['b1'].reshape(1, -1), p['g1'].reshape(1, -1),
                  p['be1'].reshape(1, -1), p['w2'], p['b2'].reshape(1, -1),
                  p['eps'].reshape(1, 1),
                  bn['g'].reshape(1, -1), bn['b'].reshape(1, -1))
        if l < L - 1:
            q = params['vn_mlps'][l]
            h_in, vn = mid_call(
                h_in, agg, segp, vn, bn1, *common,
                q['w1'], q['b1'].reshape(1, -1), q['g1'].reshape(1, -1),
                q['be1'].reshape(1, -1),
                q['w2'], q['b2'].reshape(1, -1), q['g2'].reshape(1, -1),
                q['be2'].reshape(1, -1))
        else:
            out = last_call(h_in, agg, b1n, *common,
                            params['pred_w'], params['pred_b'].reshape(1, -1))
    return out


# speedup vs baseline: 1.1557x; 1.0563x over previous
import functools

import jax
import jax.numpy as jnp
from jax import lax
from jax.experimental import pallas as pl
from jax.experimental.pallas import tpu as pltpu
from jax.experimental.pallas import tpu_sc as plsc

L = 5
G = 64
N = 10000
E = 320000
D = 128
NW = 32
NPW = 320
NPAD = NW * NPW
CHUNK = 80
IDXCAP = 11280
NLANE = 16


def _bn_n(y, g, b):
    n = y.shape[0]
    mu = jnp.mean(y, axis=0, keepdims=True)
    dd = y - mu
    dd2 = dd * dd
    var = ((jnp.sum(dd2[:n // 2], axis=0, keepdims=True)
            + jnp.sum(dd2[n // 2:], axis=0, keepdims=True)) * (1.0 / n))
    return dd * lax.rsqrt(var + 1e-5) * g + b


def _bn_g(y, g, b):
    mu = jnp.mean(y, axis=0, keepdims=True)
    dd = y - mu
    var = jnp.mean(dd * dd, axis=0, keepdims=True)
    return dd * lax.rsqrt(var + 1e-5) * g + b


def _dot(a, b):
    return jnp.dot(a, b, preferred_element_type=jnp.float32,
                   precision=lax.Precision.DEFAULT)


def _dot_hi(a, b):
    return jnp.dot(a, b, preferred_element_type=jnp.float32,
                   precision=lax.Precision.HIGHEST)


def _init_body(x_ref, v_ref, o_ref):
    o_ref[...] = x_ref[...] + v_ref[...]


def _mid_body(h_in_ref, agg_ref, segp_ref, vn_ref, bn1_ref,
              w1_ref, b1_ref, g1_ref, be1_ref, w2_ref, b2_ref, eps_ref,
              bg_ref, bb_ref,
              vw1_ref, vb1_ref, vg1_ref, vbe1_ref,
              vw2_ref, vb2_ref, vg2_ref, vbe2_ref,
              hout_ref, vnout_ref):
    h_in = h_in_ref[...]
    h = (1.0 + eps_ref[0, 0]) * h_in + agg_ref[...]
    y = _dot(h, w1_ref[...]) + b1_ref[...]
    y = jnp.maximum(_bn_n(y, g1_ref[...], be1_ref[...]), 0.0)
    z = _dot(y, w2_ref[...]) + b2_ref[...]
    hc = jnp.maximum(_bn_n(z, bg_ref[...], bb_ref[...]), 0.0)

    seg = segp_ref[0]
    for w in range(1, NW):
        seg = seg + segp_ref[w]
    vtmp = seg + vn_ref[...]
    t = _dot(vtmp, vw1_ref[...]) + vb1_ref[...]
    t = jnp.maximum(_bn_g(t, vg1_ref[...], vbe1_ref[...]), 0.0)
    t = _dot(t, vw2_ref[...]) + vb2_ref[...]
    vn_next = jnp.maximum(_bn_g(t, vg2_ref[...], vbe2_ref[...]), 0.0)
    vnout_ref[...] = vn_next

    oh_ng = (bn1_ref[...]
             == lax.broadcasted_iota(jnp.int32, (1, G), 1)).astype(jnp.float32)
    hout_ref[...] = hc + _dot_hi(oh_ng, vn_next)


def _last_body(h_in_ref, agg_ref, b1n_ref,
               w1_ref, b1_ref, g1_ref, be1_ref, w2_ref, b2_ref, eps_ref,
               bg_ref, bb_ref, pw_ref, pb_ref, out_ref):
    h_in = h_in_ref[...]
    h = (1.0 + eps_ref[0, 0]) * h_in + agg_ref[...]
    y = _dot(h, w1_ref[...]) + b1_ref[...]
    y = jnp.maximum(_bn_n(y, g1_ref[...], be1_ref[...]), 0.0)
    z = _dot(y, w2_ref[...]) + b2_ref[...]
    hc = _bn_n(z, bg_ref[...], bb_ref[...])

    oh_gn = (lax.broadcasted_iota(jnp.int32, (G, 1), 0)
             == b1n_ref[...]).astype(jnp.float32)
    counts = jnp.sum(oh_gn, axis=1, keepdims=True)
    hg = _dot_hi(oh_gn, hc) / jnp.maximum(counts, 1.0)
    out_ref[...] = _dot(hg, pw_ref[...]) + pb_ref[...]


def _make_edge_agg():
    mesh = plsc.VectorSubcoreMesh(core_axis_name="c", subcore_axis_name="s")
    f32, i32 = jnp.float32, jnp.int32

    @functools.partial(
        pl.kernel, mesh=mesh,
        compiler_params=pltpu.CompilerParams(needs_layout_passes=False),
        out_type=[jax.ShapeDtypeStruct((NPAD, D), f32),
                  jax.ShapeDtypeStruct((NW, G, D), f32)],
        scratch_types=[
            pltpu.VMEM((NPW, D), f32),
            pltpu.VMEM((G, D), f32),
            pltpu.VMEM((CHUNK, D), f32),
            pltpu.VMEM((CHUNK, D), f32),
            pltpu.VMEM((CHUNK, D), f32),
            pltpu.VMEM((CHUNK, D), f32),
            pltpu.VMEM((IDXCAP,), i32),
            pltpu.VMEM((IDXCAP,), i32),
            pltpu.VMEM((IDXCAP,), i32),
            pltpu.VMEM((CHUNK,), i32),
            pltpu.VMEM((NLANE,), i32),
            pltpu.SemaphoreType.DMA,
            pltpu.SemaphoreType.DMA,
            pltpu.SemaphoreType.DMA,
            pltpu.SemaphoreType.DMA,
        ],
    )
    def edge_agg(h_hbm, ea_hbm, srcs_hbm, perm_hbm, dsts_hbm, rp_hbm,
                 bat_hbm, agg_out, segp_out,
                 aggb, segb, hb0, eb0, hb1, eb1, sbufL, pbufL, dbufL,
                 bbuf, rpv, semh0, seme0, semh1, seme1):
        cid = lax.axis_index("c")
        sid = lax.axis_index("s")
        w = sid * 2 + cid
        ns = w * NPW
        ne = ns + NPW
        zero = jnp.zeros((NLANE,), f32)
        lanes = lax.iota(i32, NLANE)

        def zrow(r, c):
            for j in range(D // NLANE):
                aggb[r, pl.ds(j * NLANE, NLANE)] = zero
            return c
        lax.fori_loop(0, NPW, zrow, 0)

        def zrow2(r, c):
            for j in range(D // NLANE):
                segb[r, pl.ds(j * NLANE, NLANE)] = zero
            return c
        lax.fori_loop(0, G, zrow2, 0)

        def rp_at(idx):
            pltpu.sync_copy(rp_hbm.at[pl.ds(idx, NLANE)], rpv)
            return rpv[...][0]

        estart = rp_at(ns)
        eend = rp_at(ne)
        base = (estart // 8) * 8
        nchunks = lax.max((eend - base + (CHUNK - 1)) // CHUNK, 0)
        nchunks = lax.min(nchunks, IDXCAP // CHUNK)

        pltpu.sync_copy(srcs_hbm.at[pl.ds(base, IDXCAP)], sbufL)
        pltpu.sync_copy(perm_hbm.at[pl.ds(base, IDXCAP)], pbufL)
        pltpu.sync_copy(dsts_hbm.at[pl.ds(base, IDXCAP)], dbufL)

        bufs = ((hb0, eb0, semh0, seme0), (hb1, eb1, semh1, seme1))

        def start(k, b):
            hb, ebuf, semh, seme = bufs[b]
            koff = k * CHUNK
            pltpu.async_copy(h_hbm.at[sbufL.at[pl.ds(koff, CHUNK)]],
                             hb, semh)
            pltpu.async_copy(ea_hbm.at[pbufL.at[pl.ds(koff, CHUNK)]],
                             ebuf, seme)

        def wait(b):
            hb, ebuf, semh, seme = bufs[b]
            pltpu.make_async_copy(h_hbm.at[sbufL.at[pl.ds(0, CHUNK)]],
                                  hb, semh).wait()
            pltpu.make_async_copy(ea_hbm.at[pbufL.at[pl.ds(0, CHUNK)]],
                                  ebuf, seme).wait()

        def compute(k, b):
            hb, ebuf, _, _ = bufs[b]
            eb = base + k * CHUNK
            koff = k * CHUNK
            dvs, msks = [], []
            for g in range(CHUNK // NLANE):
                dv = dbufL[pl.ds(koff + g * NLANE, NLANE)] - ns
                ea_ids = eb + g * NLANE + lanes
                msk = (ea_ids >= estart) & (ea_ids < eend)
                dvs.append(dv)
                msks.append(msk)

            def fblock(fb, c2):
                for g in range(CHUNK // NLANE):
                    el = lanes + g * NLANE
                    for f2 in range(NLANE):
                        fs = jnp.full((NLANE,), fb * NLANE + f2, i32)
                        hv = plsc.load_gather(hb, [el, fs])
                        ev = plsc.load_gather(ebuf, [el, fs])
                        v = jnp.maximum(hv + ev, 0.0)
                        plsc.addupdate_scatter(aggb, [dvs[g], fs], v,
                                               mask=msks[g])
                return c2
            lax.fori_loop(0, D // NLANE, fblock, 0)

        @pl.when(nchunks > 0)
        def _():
            start(0, 0)

        def epair(p, c):
            k0 = 2 * p
            k1 = k0 + 1

            @pl.when(k1 < nchunks)
            def _():
                start(k1, 1)
            wait(0)
            compute(k0, 0)

            @pl.when(k1 + 1 < nchunks)
            def _():
                start(k1 + 1, 0)

            @pl.when(k1 < nchunks)
            def _():
                wait(1)
                compute(k1, 1)
            return c
        lax.fori_loop(0, (nchunks + 1) // 2, epair, 0)
        pltpu.sync_copy(aggb.at[pl.ds(0, NPW)], agg_out.at[pl.ds(ns, NPW)])

        sbase = (ns // 8) * 8
        for k in range(NPW // CHUNK):
            nb = sbase + k * CHUNK
            pltpu.sync_copy(bat_hbm.at[pl.ds(nb, CHUNK)], bbuf)
            pltpu.sync_copy(h_hbm.at[pl.ds(nb, CHUNK)], hb0)
            gvs, msks = [], []
            for g in range(CHUNK // NLANE):
                gv = bbuf[pl.ds(g * NLANE, NLANE)]
                n_ids = nb + g * NLANE + lanes
                msk = (n_ids >= ns) & (n_ids < ne) & (n_ids < N)
                gvs.append(gv)
                msks.append(msk)

            def sblock(fb, c2):
                for g in range(CHUNK // NLANE):
                    el = lanes + g * NLANE
                    for f2 in range(NLANE):
                        fs = jnp.full((NLANE,), fb * NLANE + f2, i32)
                        hv = plsc.load_gather(hb0, [el, fs])
                        plsc.addupdate_scatter(segb, [gvs[g], fs], hv,
                                               mask=msks[g])
                return c2
            lax.fori_loop(0, D // NLANE, sblock, 0)
        pltpu.sync_copy(segb, segp_out.at[w])

    return edge_agg


def kernel(x, edge_attr, params, edge_index, batch):
    f32, i32 = jnp.float32, jnp.int32
    T = params['pred_b'].shape[0]
    src = edge_index[0]
    dst = edge_index[1]

    perm = jnp.argsort(dst, stable=True).astype(i32)
    dst_s = dst[perm]
    src_s = src[perm]
    rowptr = jnp.searchsorted(dst_s, jnp.arange(NPAD + NLANE, dtype=i32),
                              side='left').astype(i32)
    pad_e = jnp.zeros((IDXCAP,), i32)
    src_p = jnp.concatenate([src_s, pad_e])
    perm_p = jnp.concatenate([perm, pad_e])
    dst_p = jnp.concatenate([dst_s, pad_e])
    bat_p = jnp.concatenate([batch.astype(i32),
                             jnp.zeros((NPAD + CHUNK - N,), i32)])
    bn1 = batch.reshape(N, 1)
    b1n = batch.reshape(1, N)

    init_call = pl.pallas_call(
        _init_body, out_shape=jax.ShapeDtypeStruct((N, D), f32))
    h_in = init_call(x, params['vn_emb'].reshape(1, D))
    vn = jnp.broadcast_to(params['vn_emb'], (G, D))

    edge_agg = _make_edge_agg()
    mid_call = pl.pallas_call(
        _mid_body,
        out_shape=[jax.ShapeDtypeStruct((N, D), f32),
                   jax.ShapeDtypeStruct((G, D), f32)])
    last_call = pl.pallas_call(
        _last_body, out_shape=jax.ShapeDtypeStruct((G, T), f32))

    for l in range(L):
        p = params['convs'][l]
        bn = params['bns'][l]
        h_pad = jnp.concatenate([h_in, jnp.zeros((NPAD - N, D), f32)])
        agg_pad, segp = edge_agg(h_pad, edge_attr, src_p, perm_p, dst_p,
                                 rowptr, bat_p)
        agg = agg_pad[:N]
        common = (p['w1'], p['b1'].reshape(1, -1), p['g1'].reshape(1, -1),
                  p['be1'].reshape(1, -1), p['w2'], p['b2'].reshape(1, -1),
                  p['eps'].reshape(1, 1),
                  bn['g'].reshape(1, -1), bn['b'].reshape(1, -1))
        if l < L - 1:
            q = params['vn_mlps'][l]
            h_in, vn = mid_call(
                h_in, agg, segp, vn, bn1, *common,
                q['w1'], q['b1'].reshape(1, -1), q['g1'].reshape(1, -1),
                q['be1'].reshape(1, -1),
                q['w2'], q['b2'].reshape(1, -1), q['g2'].reshape(1, -1),
                q['be2'].reshape(1, -1))
        else:
            out = last_call(h_in, agg, b1n, *common,
                            params['pred_w'], params['pred_b'].reshape(1, -1))
    return out
